# baseline (speedup 1.0000x reference)
import functools

import jax
import jax.numpy as jnp
from jax import lax
from jax.experimental import pallas as pl
from jax.experimental.pallas import tpu as pltpu
from jax.experimental.pallas import tpu_sc as plsc

BATCH = 16384
EMBED_K = 64
L = 16
NC = 2
NS = 16
NW = NC * NS
BPW = BATCH // NW
CHUNK = 128
NCH = BPW // CHUNK
IDX_ROWS_PER_W = BPW // CHUNK


def _sc_body(uidx_hbm, vidx_hbm, w_hbm, h_hbm, out_hbm,
             uidx_v, vidx_v, u_rows, v_rows, out_v, sem):
    wid = lax.axis_index("s") * NC + lax.axis_index("c")
    base = wid * BPW
    idx_row0 = wid * IDX_ROWS_PER_W

    pltpu.sync_copy(uidx_hbm.at[pl.ds(idx_row0, NCH)], uidx_v)
    pltpu.sync_copy(vidx_hbm.at[pl.ds(idx_row0, NCH)], vidx_v)

    copies = []
    for j in range(NCH):
        copies.append(pltpu.async_copy(
            w_hbm.at[uidx_v.at[j]], u_rows.at[pl.ds(j * CHUNK, CHUNK)], sem))
        copies.append(pltpu.async_copy(
            h_hbm.at[vidx_v.at[j]], v_rows.at[pl.ds(j * CHUNK, CHUNK)], sem))
    for c in copies:
        c.wait()

    lane_ids = lax.iota(jnp.int32, L)
    idx_even = (lane_ids % (L // 2)) * 2
    idx_odd = idx_even + 1
    lo_mask = lane_ids < (L // 2)

    def _perm(a, idx):
        return lax.gather(
            a, idx[:, None],
            dimension_numbers=lax.GatherDimensionNumbers(
                offset_dims=(), collapsed_slice_dims=(0,),
                start_index_map=(0,)),
            slice_sizes=(1,),
            mode=lax.GatherScatterMode.PROMISE_IN_BOUNDS)

    def _hadd(a, b):
        ce = jnp.where(lo_mask, _perm(a, idx_even), _perm(b, idx_even))
        co = jnp.where(lo_mask, _perm(a, idx_odd), _perm(b, idx_odd))
        return ce + co

    def group_body(g, _):
        vecs = []
        for k in range(L):
            r = g * L + k
            acc = u_rows[r, pl.ds(0, L)] * v_rows[r, pl.ds(0, L)]
            for j in range(1, EMBED_K // L):
                acc = acc + u_rows[r, pl.ds(j * L, L)] * v_rows[r, pl.ds(j * L, L)]
            vecs.append(acc)
        while len(vecs) > 1:
            vecs = [_hadd(vecs[i], vecs[i + 1]) for i in range(0, len(vecs), 2)]
        sums = vecs[0]
        out_v[pl.ds(g * L, L)] = 1.0 / (1.0 + jnp.exp(-sums))
        return _

    lax.fori_loop(0, BPW // L, group_body, 0, unroll=False)

    pltpu.sync_copy(out_v, out_hbm.at[pl.ds(base, BPW)])


@jax.jit
def kernel(x, W, H):
    uidx = x[:, 0].astype(jnp.int32).reshape(BATCH // CHUNK, CHUNK)
    vidx = x[:, 1].astype(jnp.int32).reshape(BATCH // CHUNK, CHUNK)
    mesh = plsc.VectorSubcoreMesh(core_axis_name="c", subcore_axis_name="s")
    f = functools.partial(
        pl.kernel, mesh=mesh,
        compiler_params=pltpu.CompilerParams(use_tc_tiling_on_sc=False),
        out_type=jax.ShapeDtypeStruct((BATCH,), jnp.float32),
        scratch_types=[
            pltpu.VMEM((NCH, CHUNK), jnp.int32),
            pltpu.VMEM((NCH, CHUNK), jnp.int32),
            pltpu.VMEM((BPW, EMBED_K), jnp.float32),
            pltpu.VMEM((BPW, EMBED_K), jnp.float32),
            pltpu.VMEM((BPW,), jnp.float32),
            pltpu.SemaphoreType.DMA,
        ],
    )(_sc_body)
    return f(uidx, vidx, W, H)
